# baseline (speedup 1.0000x reference)
import functools

import numpy as np
import jax
import jax.numpy as jnp
from jax import lax
from jax.experimental import pallas as pl
from jax.experimental.pallas import tpu as pltpu
from jax.experimental.pallas import tpu_sc as plsc

_H = 40
_W = 40
_HW = _H * _W
_C = 512
_B = 8
_PC_LO_X = -51.2
_PC_LO_Y = -51.2
_DIM_X = 102.4
_DIM_Y = 102.4
_CELL_X = np.float32(_DIM_X / _W)
_CELL_Y = np.float32(_DIM_Y / _H)
_RATIO_LO = 1.0
_RATIO_HI = 6.0

_NC = 2
_NS = 16
_L = 16
_NWORK = _NC * _NS
_ROWS = _B * _HW
_RPW = _ROWS // _NWORK
_CHUNK = 80
_NCHUNK = _RPW // _CHUNK

_ww, _hh = np.meshgrid(range(_W), range(_H))
_wwf = (_ww.reshape(-1).astype(np.float64) + 0.5) / _W * _DIM_X + _PC_LO_X
_hhf = (_hh.reshape(-1).astype(np.float64) + 0.5) / _H * _DIM_Y + _PC_LO_Y
_GRIDS = np.stack([_wwf, _hhf], 1).astype(np.float32)


def _var_sc_body(a_hbm, v_hbm, buf, s1b, s2b, vout):
    c = lax.axis_index("c")
    s = lax.axis_index("s")
    wid = s * jnp.int32(_NC) + c
    base = wid * jnp.int32(_RPW)
    for ch in range(_NCHUNK):
        row0 = base + jnp.int32(ch * _CHUNK)
        pltpu.sync_copy(a_hbm.at[pl.ds(row0, _CHUNK)], buf)

        def _row(r, carry):
            x = buf[r, pl.ds(0, _L)]
            acc1 = x
            acc2 = x * x
            for j in range(1, _C // _L):
                xv = buf[r, pl.ds(j * _L, _L)]
                acc1 = acc1 + xv
                acc2 = acc2 + xv * xv
            rb = r * jnp.int32(_L)
            s1b[pl.ds(rb, _L)] = acc1
            s2b[pl.ds(rb, _L)] = acc2
            return carry

        lax.fori_loop(jnp.int32(0), jnp.int32(_CHUNK), _row, jnp.int32(0))

        for g in range(_CHUNK // _L):
            ridx = (lax.iota(jnp.int32, _L) + jnp.int32(g * _L)) * jnp.int32(_L)
            r1 = jnp.zeros((_L,), jnp.float32)
            r2 = jnp.zeros((_L,), jnp.float32)
            for j in range(_L):
                idx = ridx + jnp.int32(j)
                r1 = r1 + plsc.load_gather(s1b, [idx])
                r2 = r2 + plsc.load_gather(s2b, [idx])
            v = (r2 - r1 * r1 * (1.0 / _C)) * (1.0 / (_C - 1))
            vout[pl.ds(g * _L, _L)] = v
        pltpu.sync_copy(vout, v_hbm.at[pl.ds(row0, _CHUNK)])


def _var_sc(atten_flat):
    mesh = plsc.VectorSubcoreMesh(core_axis_name="c", subcore_axis_name="s")
    run = functools.partial(
        pl.kernel,
        mesh=mesh,
        compiler_params=pltpu.CompilerParams(needs_layout_passes=False),
        out_type=jax.ShapeDtypeStruct((_ROWS,), jnp.float32),
        scratch_types=[
            pltpu.VMEM((_CHUNK, _C), jnp.float32),
            pltpu.VMEM((_CHUNK * _L,), jnp.float32),
            pltpu.VMEM((_CHUNK * _L,), jnp.float32),
            pltpu.VMEM((_CHUNK,), jnp.float32),
        ],
    )(_var_sc_body)
    return run(atten_flat)


def _loss_body(v_ref, g_ref, gr_ref, out_ref, acc_ref):
    b = pl.program_id(0)

    @pl.when(b == 0)
    def _init():
        acc_ref[0] = 0.0
        acc_ref[1] = 0.0

    v = v_ref[0]

    g = g_ref[0]
    cx = g[0:1]
    cy = g[1:2]
    dl = g[3:4]
    dw = g[4:5]
    yaw = g[6:7]
    rl = jnp.clip(_CELL_X / dl, _RATIO_LO, _RATIO_HI)
    rw = jnp.clip(_CELL_Y / dw, _RATIO_LO, _RATIO_HI)
    hx = 0.5 * dl * rl
    hy = 0.5 * dw * rw
    sn = jnp.sin(yaw)
    cs = jnp.cos(yaw)
    xs = []
    ys = []
    for sx, sy in ((-1.0, -1.0), (-1.0, 1.0), (1.0, 1.0), (1.0, -1.0)):
        lx = sx * hx
        ly = sy * hy
        xs.append(lx * cs - ly * sn + cx)
        ys.append(lx * sn + ly * cs + cy)

    gx = gr_ref[:, 0:1]
    gy = gr_ref[:, 1:2]
    all_ge = None
    all_le = None
    for k in range(4):
        kn = (k + 1) % 4
        ex = xs[kn] - xs[k]
        ey = ys[kn] - ys[k]
        cross = ex * (gy - ys[k]) - ey * (gx - xs[k])
        ge = cross >= 0.0
        le = cross <= 0.0
        all_ge = ge if all_ge is None else (all_ge & ge)
        all_le = le if all_le is None else (all_le & le)
    inside = all_ge | all_le

    d2 = (gx - cx) ** 2 + (gy - cy) ** 2
    mind = jnp.min(d2, axis=0, keepdims=True)
    cellid = jax.lax.broadcasted_iota(jnp.int32, (_HW, 32), 0)
    cand = jnp.where(d2 == mind, cellid, _HW)
    mi = jnp.min(cand, axis=0, keepdims=True)
    pos = inside | (cellid == mi)

    ki = jnp.sum(pos.astype(jnp.int32), axis=1, keepdims=True,
                 dtype=jnp.int32)
    boxid = jax.lax.broadcasted_iota(jnp.int32, (_HW, 32), 1)
    lastc = jnp.max(jnp.where(pos, boxid, -1), axis=1, keepdims=True)
    own = ((ki & 1) == 1) & (boxid == lastc)

    ownf = own.astype(jnp.float32)
    cnt = jnp.sum(ownf, axis=0, keepdims=True)
    vs = jnp.sum(ownf * v, axis=0, keepdims=True)
    has = cnt > 0.0
    contrib = jnp.where(has, vs / jnp.maximum(cnt, 1.0), 0.0)
    acc_ref[0] += -jnp.sum(contrib)
    acc_ref[1] += jnp.sum(has.astype(jnp.float32))

    @pl.when(b == pl.num_programs(0) - 1)
    def _fin():
        out_ref[0, 0] = acc_ref[0] / jnp.maximum(acc_ref[1], 1.0)


def _loss_tc(vmat, gtT, grids):
    out = pl.pallas_call(
        _loss_body,
        grid=(_B,),
        in_specs=[
            pl.BlockSpec((1, _HW, 1), lambda b: (b, b * 0, b * 0)),
            pl.BlockSpec((1, 7, 32), lambda b: (b, b * 0, b * 0)),
            pl.BlockSpec((_HW, 2), lambda b: (b * 0, b * 0)),
        ],
        out_specs=pl.BlockSpec((1, 1), lambda b: (b * 0, b * 0),
                               memory_space=pltpu.SMEM),
        out_shape=jax.ShapeDtypeStruct((1, 1), jnp.float32),
        scratch_shapes=[pltpu.SMEM((2,), jnp.float32)],
    )(vmat, gtT, grids)
    return out[0, 0]


def kernel(atten_map, gt_bboxes):
    atten_flat = atten_map.reshape(_ROWS, _C)
    v = _var_sc(atten_flat)
    vmat = v.reshape(_B, _HW, 1)
    gtT = jnp.transpose(gt_bboxes.astype(jnp.float32), (0, 2, 1))
    grids = jnp.asarray(_GRIDS)
    return _loss_tc(vmat, gtT, grids)


# speedup vs baseline: 3.0547x; 3.0547x over previous
import numpy as np
import jax
import jax.numpy as jnp
from jax.experimental import pallas as pl
from jax.experimental.pallas import tpu as pltpu

_H = 40
_W = 40
_HW = _H * _W
_C = 512
_B = 8
_M = 32
_BM = _B * _M
_CB = 2
_RB = _HW // _CB
_PC_LO_X = -51.2
_PC_LO_Y = -51.2
_DIM_X = 102.4
_DIM_Y = 102.4
_CELL_X = np.float32(_DIM_X / _W)
_CELL_Y = np.float32(_DIM_Y / _H)
_RATIO_LO = 1.0
_RATIO_HI = 6.0

_ww, _hh = np.meshgrid(range(_W), range(_H))
_wwf = (_ww.reshape(-1).astype(np.float64) + 0.5) / _W * _DIM_X + _PC_LO_X
_hhf = (_hh.reshape(-1).astype(np.float64) + 0.5) / _H * _DIM_Y + _PC_LO_Y
_GRIDS = np.stack([_wwf, _hhf], 1).astype(np.float32)

_W_SAME = np.zeros((_BM, _BM), np.float32)
_W_GT = np.zeros((_BM, _BM), np.float32)
for _b in range(_B):
    _W_SAME[_b * _M:(_b + 1) * _M, _b * _M:(_b + 1) * _M] = 1.0
    _W_GT[_b * _M:(_b + 1) * _M, _b * _M:(_b + 1) * _M] = np.triu(
        np.ones((_M, _M), np.float32), 1)
_E_BCAST = np.zeros((_B, _BM), np.float32)
for _b in range(_B):
    _E_BCAST[_b, _b * _M:(_b + 1) * _M] = 1.0


def _var_body(a_ref, v_ref):
    x = a_ref[0]
    s1 = jnp.sum(x, axis=1, keepdims=True)
    s2 = jnp.sum(x * x, axis=1, keepdims=True)
    v = (s2 - s1 * s1 * (1.0 / _C)) * (1.0 / (_C - 1))
    v_ref[...] = jnp.reshape(v, (1, _HW, 1))


def _var_tc(atten_map):
    return pl.pallas_call(
        _var_body,
        grid=(_B,),
        in_specs=[
            pl.BlockSpec((1, _HW, _C), lambda b: (b, b * 0, b * 0)),
        ],
        out_specs=pl.BlockSpec((1, _HW, 1), lambda b: (b, b * 0, b * 0)),
        out_shape=jax.ShapeDtypeStruct((_B, _HW, 1), jnp.float32),
    )(atten_map)


def _loss_body(vt_ref, p_ref, gr_ref, ws_ref, wg_ref, e_ref, out_ref):
    p = p_ref[...]
    cx = p[0:1]
    cy = p[1:2]
    dl = p[3:4]
    dw = p[4:5]
    yaw = p[6:7]
    rl = jnp.clip(_CELL_X / dl, _RATIO_LO, _RATIO_HI)
    rw = jnp.clip(_CELL_Y / dw, _RATIO_LO, _RATIO_HI)
    hx = 0.5 * dl * rl
    hy = 0.5 * dw * rw
    sn = jnp.sin(yaw)
    cs = jnp.cos(yaw)
    xs = []
    ys = []
    for sx, sy in ((-1.0, -1.0), (-1.0, 1.0), (1.0, 1.0), (1.0, -1.0)):
        lx = sx * hx
        ly = sy * hy
        xs.append(lx * cs - ly * sn + cx)
        ys.append(lx * sn + ly * cs + cy)

    gx = gr_ref[:, 0:1]
    gy = gr_ref[:, 1:2]
    all_ge = None
    all_le = None
    for k in range(4):
        kn = (k + 1) % 4
        ex = xs[kn] - xs[k]
        ey = ys[kn] - ys[k]
        cross = ex * (gy - ys[k]) - ey * (gx - xs[k])
        ge = cross >= 0.0
        le = cross <= 0.0
        all_ge = ge if all_ge is None else (all_ge & ge)
        all_le = le if all_le is None else (all_le & le)
    inside = all_ge | all_le

    d2 = (gx - cx) ** 2 + (gy - cy) ** 2
    mind = jnp.min(d2, axis=0, keepdims=True)
    cellf = jax.lax.broadcasted_iota(
        jnp.int32, (_HW, _BM), 0).astype(jnp.float32)
    cand = jnp.where(d2 == mind, cellf, float(_HW))
    mi = jnp.min(cand, axis=0, keepdims=True)
    posf = (inside | (cellf == mi)).astype(jnp.float32)

    kib = jax.lax.dot(posf, ws_ref[...],
                      precision=jax.lax.Precision.HIGHEST)
    cgt = jax.lax.dot(posf, wg_ref[...],
                      precision=jax.lax.Precision.HIGHEST)
    odd = (kib - jnp.floor(kib * 0.5) * 2.0) == 1.0
    own = posf * (odd & (cgt == 0.0)).astype(jnp.float32)

    vrep = jax.lax.dot_general(
        vt_ref[...], e_ref[...], (((0,), (0,)), ((), ())),
        precision=jax.lax.Precision.HIGHEST)

    cnt = jnp.sum(own, axis=0, keepdims=True)
    vs = jnp.sum(own * vrep, axis=0, keepdims=True)
    has = cnt > 0.0
    contrib = jnp.where(has, vs / jnp.maximum(cnt, 1.0), 0.0)
    loss = -jnp.sum(contrib)
    posn = jnp.sum(has.astype(jnp.float32))
    out_ref[0, 0] = loss / jnp.maximum(posn, 1.0)


def _loss_tc(vt, params, grids, ws, wg, eb):
    out = pl.pallas_call(
        _loss_body,
        out_specs=pl.BlockSpec(memory_space=pltpu.SMEM),
        out_shape=jax.ShapeDtypeStruct((1, 1), jnp.float32),
    )(vt, params, grids, ws, wg, eb)
    return out[0, 0]


def kernel(atten_map, gt_bboxes):
    vb = _var_tc(atten_map).reshape(_B, _HW)
    params = jnp.transpose(gt_bboxes.astype(jnp.float32),
                           (2, 0, 1)).reshape(7, _BM)
    grids = jnp.asarray(_GRIDS)
    return _loss_tc(vb, params, grids, jnp.asarray(_W_SAME),
                    jnp.asarray(_W_GT), jnp.asarray(_E_BCAST))
